# Initial kernel scaffold; baseline (speedup 1.0000x reference)
import jax
import jax.numpy as jnp
from jax.experimental import pallas as pl


def kernel(x, edge_index, edge_attr, params):
    raise NotImplementedError("write your pallas kernel here")


# speedup vs baseline: 4.0444x; 4.0444x over previous
import functools

import jax
import jax.numpy as jnp
from jax import lax
from jax.experimental import pallas as pl
from jax.experimental.pallas import tpu as pltpu
from jax.experimental.pallas import tpu_sc as plsc

F32 = jnp.float32
FPAD = 48
CH = 128
NW = 32
BE = 1024
BN = 1024

@functools.lru_cache(maxsize=None)
def _mesh():
    return plsc.VectorSubcoreMesh(core_axis_name="c", subcore_axis_name="s")


def _sc_gather(xp, srcp, dstp):
    ep = srcp.shape[0]
    per_tile = ep // NW
    nch = per_tile // CH

    @functools.partial(
        pl.kernel, mesh=_mesh(),
        compiler_params=pltpu.CompilerParams(use_tc_tiling_on_sc=False),
        out_type=(jax.ShapeDtypeStruct((ep, FPAD), F32),
                  jax.ShapeDtypeStruct((ep, FPAD), F32)),
        scratch_types=[pltpu.VMEM((CH,), jnp.int32),
                       pltpu.VMEM((CH, FPAD), F32),
                       pltpu.VMEM((CH,), jnp.int32),
                       pltpu.VMEM((CH, FPAD), F32),
                       pltpu.SemaphoreType.DMA,
                       pltpu.SemaphoreType.DMA])
    def gk(x_hbm, s_hbm, d_hbm, xj_hbm, xi_hbm, si_v, sr_v, di_v, dr_v, sem1, sem2):
        wid = lax.axis_index("s") * 2 + lax.axis_index("c")

        @pl.loop(0, nch)
        def _(i):
            base = wid * per_tile + i * CH
            pltpu.sync_copy(s_hbm.at[pl.ds(base, CH)], si_v)
            pltpu.sync_copy(d_hbm.at[pl.ds(base, CH)], di_v)
            cp1 = pltpu.async_copy(x_hbm.at[si_v], sr_v, sem1)
            cp2 = pltpu.async_copy(x_hbm.at[di_v], dr_v, sem2)
            cp1.wait()
            cp2.wait()
            pltpu.sync_copy(sr_v, xj_hbm.at[pl.ds(base, CH)])
            pltpu.sync_copy(dr_v, xi_hbm.at[pl.ds(base, CH)])

    return gk(xp, srcp, dstp)


def _sc_scatter(payload, dstp, zeros_acc):
    ep = payload.shape[0]
    np_ = zeros_acc.shape[0]
    per_tile = ep // NW
    nch = per_tile // CH
    rows = np_ // 16

    @functools.partial(
        pl.kernel, mesh=_mesh(),
        compiler_params=pltpu.CompilerParams(use_tc_tiling_on_sc=False),
        out_type=jax.ShapeDtypeStruct((2, np_, FPAD), F32),
        scratch_types=[pltpu.VMEM((CH, FPAD), F32),
                       pltpu.VMEM((CH,), jnp.int32),
                       pltpu.VMEM_SHARED((np_, FPAD), F32)])
    def sk(pay_hbm, d_hbm, z_hbm, out_hbm, pay_v, idx_v, acc_sh):
        cid = lax.axis_index("c")
        sid = lax.axis_index("s")
        pltpu.sync_copy(z_hbm.at[pl.ds(sid * rows, rows)],
                        acc_sh.at[pl.ds(sid * rows, rows)])
        plsc.subcore_barrier()
        wid = sid * 2 + cid

        @pl.loop(0, nch)
        def _(i):
            base = wid * per_tile + i * CH
            pltpu.sync_copy(d_hbm.at[pl.ds(base, CH)], idx_v)
            pltpu.sync_copy(pay_hbm.at[pl.ds(base, CH)], pay_v)
            pltpu.sync_copy(pay_v, acc_sh.at[idx_v], add=True)

        plsc.subcore_barrier()
        pltpu.sync_copy(acc_sh.at[pl.ds(sid * rows, rows)],
                        out_hbm.at[cid, pl.ds(sid * rows, rows)])

    return sk(payload, dstp, zeros_acc)


BF16 = jnp.bfloat16


def _dot(a, b):
    return jnp.dot(a, b, preferred_element_type=F32)


def _split(a):
    ah = a.astype(BF16)
    al = (a - ah.astype(F32)).astype(BF16)
    return ah, al


def _dot3(a, wh, wl):
    ah, al = _split(a)
    return _dot(ah, wh) + _dot(al, wh) + _dot(ah, wl)


def _dot_b(a, wb):
    return _dot(a.astype(BF16), wb)


def _dot_x(a, wb):
    ah, al = _split(a)
    return _dot(ah, wb) + _dot(al, wb)


def _mlp(h, nxt, n_layers):
    for i in range(n_layers):
        h = _dot_x(h, nxt()) + nxt()
        if i < n_layers - 1:
            h = jnp.maximum(h, 0.0)
    return h


def _hyper(xin, c, nxt):
    w3mb, wb2h, wb2l, bwb, bb, rexp, sfin = (
        nxt(), nxt(), nxt(), nxt(), nxt(), nxt(), nxt())
    u = _dot3(xin, wb2h, wb2l) + _dot_x(c, bwb) + bb
    a = _dot_x(xin, w3mb)
    aw = a * _dot_x(c, rexp)
    s = aw[:, 0:512] + aw[:, 512:1024]
    s = s[:, 0:256] + s[:, 256:512]
    s = s[:, 0:128] + s[:, 128:256]
    return u + _dot_x(s, sfin)


def _tc_edge(weights, xj, xi, ea, nf):
    ep = xj.shape[0]
    efd = ea.shape[1]
    n_w = len(weights)

    def body(*refs):
        xj_ref, xi_ref, ea_ref = refs[0:3]
        wrefs = refs[3:3 + n_w]
        hp_ref, z_ref = refs[3 + n_w:]
        it = iter(wrefs)
        nxt = lambda: next(it)[...]
        xjb = xj_ref[...]
        kj = xjb[:, 0:6]
        hj = xjb[:, 6:6 + nf]
        hi = xi_ref[...][:, 6:6 + nf]
        c = _mlp(kj, nxt, 3)
        xin = jnp.concatenate([hi, hj, ea_ref[...]], axis=1)
        u = _hyper(xin, c, nxt)
        ha = _mlp(jnp.concatenate([u, u], axis=1), nxt, 4)
        hp_ref[...] = ha[:, 0:32]
        z_ref[...] = ha[:, 32:33]

    same = lambda i: (i, 0)
    fixed = lambda i: (0, 0)
    in_specs = ([pl.BlockSpec((BE, FPAD), same), pl.BlockSpec((BE, FPAD), same),
                 pl.BlockSpec((BE, efd), same)]
                + [pl.BlockSpec(w.shape, fixed) for w in weights])
    return pl.pallas_call(
        body, grid=(ep // BE,), in_specs=in_specs,
        out_specs=[pl.BlockSpec((BE, 32), same), pl.BlockSpec((BE, 1), same)],
        out_shape=[jax.ShapeDtypeStruct((ep, 32), F32),
                   jax.ShapeDtypeStruct((ep, 1), F32)],
        compiler_params=pltpu.CompilerParams(
            dimension_semantics=("arbitrary",)),
    )(xj, xi, ea, *weights)


def _tc_max(z):
    def body(z_ref, o_ref):
        o_ref[0, 0] = jnp.max(z_ref[...])

    return pl.pallas_call(
        body,
        out_specs=pl.BlockSpec(memory_space=pltpu.SMEM),
        out_shape=jax.ShapeDtypeStruct((1, 1), F32))(z)


def _tc_payload(z, hp, zmax):
    ep = z.shape[0]

    def body(z_ref, hp_ref, m_ref, o_ref):
        e = jnp.exp(z_ref[...] - m_ref[0, 0])
        o_ref[...] = jnp.concatenate(
            [hp_ref[...] * e, e, jnp.zeros((BE, FPAD - 33), F32)], axis=1)

    same = lambda i: (i, 0)
    return pl.pallas_call(
        body, grid=(ep // BE,),
        in_specs=[pl.BlockSpec((BE, 1), same), pl.BlockSpec((BE, 32), same),
                  pl.BlockSpec(memory_space=pltpu.SMEM)],
        out_specs=pl.BlockSpec((BE, FPAD), same),
        out_shape=jax.ShapeDtypeStruct((ep, FPAD), F32),
        compiler_params=pltpu.CompilerParams(
            dimension_semantics=("arbitrary",)),
    )(z, hp, zmax)


def _tc_node(weights, xp, p0, p1, nf):
    np_ = xp.shape[0]
    n_w = len(weights)

    def body(*refs):
        x_ref, p0_ref, p1_ref = refs[0:3]
        wrefs = refs[3:3 + n_w]
        o_ref = refs[3 + n_w]
        it = iter(wrefs)
        nxt = lambda: next(it)[...]
        s = p0_ref[...] + p1_ref[...]
        agg = s[:, 0:32] / (s[:, 32:33] + 1e-30)
        xb = x_ref[...]
        kn = xb[:, 0:6]
        h = xb[:, 6:6 + nf]
        c = _mlp(kn, nxt, 3)
        u = _hyper(agg, c, nxt)
        hp = _mlp(jnp.concatenate([h, u], axis=1), nxt, 4)
        o_ref[...] = jnp.concatenate(
            [kn, hp, jnp.zeros((BN, FPAD - 38), F32)], axis=1)

    same = lambda i: (i, 0)
    fixed = lambda i: (0, 0)
    in_specs = ([pl.BlockSpec((BN, FPAD), same)] * 3
                + [pl.BlockSpec(w.shape, fixed) for w in weights])
    return pl.pallas_call(
        body, grid=(np_ // BN,), in_specs=in_specs,
        out_specs=pl.BlockSpec((BN, FPAD), same),
        out_shape=jax.ShapeDtypeStruct((np_, FPAD), F32),
        compiler_params=pltpu.CompilerParams(
            dimension_semantics=("arbitrary",)),
    )(xp, p0, p1, *weights)


def _lin2(wb):
    w, b = wb
    return [w.astype(BF16), b.reshape(1, -1)]


def _mi_group(mi, d):
    rexp = jnp.kron(jnp.eye(32, dtype=BF16), jnp.ones((1, 32), BF16))
    sfin = jnp.kron(jnp.ones((4, 1), BF16), jnp.eye(32, dtype=BF16))
    ww, wb = mi['W']
    bw, bb = mi['b']
    w3m = ww.reshape(32, d, 32).transpose(1, 0, 2).reshape(d, 1024)
    wb2h, wb2l = _split(wb.reshape(d, 32))
    return [w3m.astype(BF16), wb2h, wb2l, bw.astype(BF16),
            bb.reshape(1, 32), rexp, sfin]


def _blockdiag2(ta, tb):
    wa, ba = ta
    wb, bb = tb
    ia, oa = wa.shape
    ib, ob = wb.shape
    w = jnp.zeros((ia + ib, oa + ob), F32)
    w = w.at[:ia, :oa].set(wa).at[ia:, oa:].set(wb)
    return (w, jnp.concatenate([ba, bb]))


def _prep_params(p, nf, ef):
    d = 2 * nf + ef
    ea_fused = [_blockdiag2(ta, tb) for ta, tb in zip(p['edge'], p['attn'])]
    edge_ws = (sum((_lin2(t) for t in p['etype']), [])
               + _mi_group(p['mi_edge'], d)
               + sum((_lin2(t) for t in ea_fused), []))
    node_ws = (sum((_lin2(t) for t in p['ntype']), [])
               + _mi_group(p['mi_node'], 32)
               + sum((_lin2(t) for t in p['node']), []))
    return edge_ws, node_ws


_DIAG_JNP_GATHER = False
_DIAG_JNP_SCATTER = False


def _tga_layer(p, xp, srcp, dstp, eap, zeros_acc, nf, ef):
    edge_ws, node_ws = _prep_params(p, nf, ef)
    if _DIAG_JNP_GATHER:
        xj, xi = xp[srcp], xp[dstp]
    else:
        xj, xi = _sc_gather(xp, srcp, dstp)
    hp_e, z = _tc_edge(edge_ws, xj, xi, eap, nf)
    zmax = _tc_max(z)
    payload = _tc_payload(z, hp_e, zmax)
    if _DIAG_JNP_SCATTER:
        seg = jax.ops.segment_sum(payload, dstp,
                                  num_segments=zeros_acc.shape[0])
        parts = jnp.stack([seg, jnp.zeros_like(seg)])
    else:
        parts = _sc_scatter(payload, dstp, zeros_acc)
    x_new = _tc_node(node_ws, xp, parts[0], parts[1], nf)
    return x_new, hp_e


def kernel(x, edge_index, edge_attr, params):
    n, f0 = x.shape
    e = edge_index.shape[1]
    ep = pl.cdiv(e, NW * CH) * NW * CH
    np_ = pl.cdiv(n + 1, BN) * BN

    src = edge_index[0]
    dst = edge_index[1]
    pad_e = ep - e
    pad_idx = (n + jnp.arange(pad_e, dtype=jnp.int32) % (np_ - n)).astype(jnp.int32)
    srcp = jnp.concatenate([src, pad_idx])
    dstp = jnp.concatenate([dst, pad_idx])
    eap = jnp.concatenate(
        [edge_attr, jnp.zeros((pad_e, edge_attr.shape[1]), F32)], axis=0)
    xp = jnp.zeros((np_, FPAD), F32).at[:n, :f0].set(x)
    zeros_acc = jnp.zeros((np_, FPAD), F32)

    x1p, ea1p = _tga_layer(params['l1'], xp, srcp, dstp, eap, zeros_acc, 11, 1)
    x2p, ea2p = _tga_layer(params['l2'], x1p, srcp, dstp, ea1p, zeros_acc, 32, 32)
    return (x2p[:n, 6:38], edge_index, ea2p[:e])
